# baseline (speedup 1.0000x reference)
import functools

import jax
import jax.numpy as jnp
from jax.experimental import pallas as pl
from jax.experimental.pallas import tpu as pltpu
from jax.experimental.pallas import tpu_sc as plsc

V = 50265
D = 1024
E = 8
K = 5
DFF = 1024
B = 1
S = 2048

BF = jnp.bfloat16
F32 = jnp.float32

_NC = 2
_NS = 16
_NW = _NC * _NS


def _sc_row_gather(table, idx, n_split):
    n, d = len(idx), table.shape[1]
    bpw = n // _NW
    sub = bpw // n_split
    mesh = plsc.VectorSubcoreMesh(core_axis_name="c", subcore_axis_name="s")

    @functools.partial(
        pl.kernel,
        mesh=mesh,
        out_type=jax.ShapeDtypeStruct((n, d), table.dtype),
        scratch_types=[
            pltpu.VMEM((sub,), jnp.int32),
            pltpu.VMEM((sub, d), table.dtype),
            pltpu.SemaphoreType.DMA,
        ],
    )
    def k(tab_hbm, idx_hbm, out_hbm, idx_v, rows_v, sem):
        wid = jax.lax.axis_index("s") * _NC + jax.lax.axis_index("c")

        @pl.loop(0, n_split)
        def _(i):
            base = wid * bpw + i * sub
            pltpu.sync_copy(idx_hbm.at[pl.ds(base, sub)], idx_v)
            pltpu.async_copy(tab_hbm.at[idx_v], rows_v, sem).wait()
            pltpu.sync_copy(rows_v, out_hbm.at[pl.ds(base, sub)])

    return k(table, idx)


def _embed_body(g_ref, p_ref, t_ref, gam_ref, bet_ref, o_ref):
    h = g_ref[...] + p_ref[...] + t_ref[...]
    m = jnp.mean(h, axis=-1, keepdims=True)
    v = jnp.mean((h - m) ** 2, axis=-1, keepdims=True)
    x = (h - m) / jnp.sqrt(v + 1e-5) * gam_ref[...] + bet_ref[...]
    o_ref[...] = x.astype(BF)


def _embed_finish(gathered, pos, type_emb, gamma, beta):
    tb = 256
    return pl.pallas_call(
        _embed_body,
        grid=(S // tb,),
        in_specs=[
            pl.BlockSpec((tb, D), lambda i: (i, 0)),
            pl.BlockSpec((tb, D), lambda i: (i, 0)),
            pl.BlockSpec((1, D), lambda i: (0, 0)),
            pl.BlockSpec((1, D), lambda i: (0, 0)),
            pl.BlockSpec((1, D), lambda i: (0, 0)),
        ],
        out_specs=pl.BlockSpec((tb, D), lambda i: (i, 0)),
        out_shape=jax.ShapeDtypeStruct((S, D), BF),
    )(gathered, pos, type_emb, gamma.reshape(1, D), beta.reshape(1, D))


def _moe_body(x_ref, gate_ref, w1_ref, b1_ref, w2_ref, b2_ref, o_ref,
              wsel_ref, acc_ref):
    e = pl.program_id(0)

    @pl.when(e == 0)
    def _():
        l = jnp.dot(x_ref[...], gate_ref[...], preferred_element_type=F32)
        rank = jnp.zeros((S, E), dtype=jnp.int32)
        eidx = jax.lax.broadcasted_iota(jnp.int32, (S, E), 1)
        for j in range(E):
            lj = l[:, j:j + 1]
            beats = (lj > l) | ((lj == l) & (j < eidx))
            rank = rank + beats.astype(jnp.int32)
        sel = rank < K
        lm = jnp.where(sel, l, -jnp.inf)
        mx = jnp.max(lm, axis=1, keepdims=True)
        ex = jnp.where(sel, jnp.exp(l - mx), 0.0)
        wsel_ref[...] = (ex / jnp.sum(ex, axis=1, keepdims=True)).astype(BF)
        acc_ref[...] = jnp.zeros_like(acc_ref)

    ch = 512
    for c in range(S // ch):
        sl = pl.ds(c * ch, ch)
        h = jnp.dot(x_ref[sl, :], w1_ref[0], preferred_element_type=F32)
        h = jax.nn.gelu(h + b1_ref[0])
        y = jnp.dot(h, w2_ref[0], preferred_element_type=F32)
        y = (y + b2_ref[0]).astype(BF).astype(F32)
        eidx2 = jax.lax.broadcasted_iota(jnp.int32, (ch, E), 1)
        w_e = jnp.sum(
            jnp.where(eidx2 == e, wsel_ref[sl, :].astype(F32), 0.0),
            axis=1, keepdims=True)
        acc_ref[sl, :] += y * w_e

    @pl.when(e == E - 1)
    def _():
        o_ref[...] = acc_ref[...].astype(BF)


def _moe_block(x, gate, w1, b1, w2, b2):
    return pl.pallas_call(
        _moe_body,
        grid=(E,),
        in_specs=[
            pl.BlockSpec((S, D), lambda e: (0, 0)),
            pl.BlockSpec((D, E), lambda e: (0, 0)),
            pl.BlockSpec((1, D, DFF), lambda e: (e, 0, 0)),
            pl.BlockSpec((1, 1, DFF), lambda e: (e, 0, 0)),
            pl.BlockSpec((1, DFF, D), lambda e: (e, 0, 0)),
            pl.BlockSpec((1, 1, D), lambda e: (e, 0, 0)),
        ],
        out_specs=pl.BlockSpec((S, D), lambda e: (0, 0)),
        out_shape=jax.ShapeDtypeStruct((S, D), BF),
        scratch_shapes=[
            pltpu.VMEM((S, E), BF),
            pltpu.VMEM((S, D), F32),
        ],
    )(x, gate, w1, b1.reshape(E, 1, DFF), w2, b2.reshape(E, 1, D))


_VT = 1024
_NV = (V + _VT - 1) // _VT


def _lm_body(x_ref, w_ref, b_ref, sc_ref, m_out, s_out, m_ref, s_ref):
    v = pl.program_id(0)
    ch = 512

    def tile(c, masked):
        sl = pl.ds(c * ch, ch)
        sc = jax.lax.dot_general(
            w_ref[...], x_ref[sl, :],
            dimension_numbers=(((0,), (1,)), ((), ())),
            preferred_element_type=F32)
        sc = sc + b_ref[...]
        sc_ref[:, sl] = sc
        if masked:
            row = jax.lax.broadcasted_iota(jnp.int32, (_VT, ch), 0) + v * _VT
            scm = jnp.where(row < V, sc, -jnp.inf)
        else:
            scm = sc
        tmax = jnp.max(scm, axis=0, keepdims=True)

        @pl.when(v == 0)
        def _():
            m_ref[:, sl] = tmax
            s_ref[:, sl] = jnp.sum(jnp.exp(scm - tmax), axis=0, keepdims=True)

        @pl.when(v > 0)
        def _():
            m_old = m_ref[:, sl]
            m_new = jnp.maximum(m_old, tmax)
            s_ref[:, sl] = (s_ref[:, sl] * jnp.exp(m_old - m_new)
                            + jnp.sum(jnp.exp(scm - m_new), axis=0,
                                      keepdims=True))
            m_ref[:, sl] = m_new

    @pl.when(v < _NV - 1)
    def _():
        for c in range(S // ch):
            tile(c, masked=False)

    @pl.when(v == _NV - 1)
    def _():
        for c in range(S // ch):
            tile(c, masked=True)
        m_out[...] = m_ref[...]
        s_out[...] = s_ref[...]


def _lm_head(x, lm_w, lm_b):
    return pl.pallas_call(
        _lm_body,
        grid=(_NV,),
        in_specs=[
            pl.BlockSpec((S, D), lambda v: (0, 0)),
            pl.BlockSpec((D, _VT), lambda v: (0, v)),
            pl.BlockSpec((_VT, 1), lambda v: (v, 0)),
        ],
        out_specs=[
            pl.BlockSpec((_VT, S), lambda v: (v, 0)),
            pl.BlockSpec((1, S), lambda v: (0, 0)),
            pl.BlockSpec((1, S), lambda v: (0, 0)),
        ],
        out_shape=[
            jax.ShapeDtypeStruct((V, S), F32),
            jax.ShapeDtypeStruct((1, S), F32),
            jax.ShapeDtypeStruct((1, S), F32),
        ],
        scratch_shapes=[
            pltpu.VMEM((1, S), F32),
            pltpu.VMEM((1, S), F32),
        ],
    )(x, lm_w, lm_b.reshape(V, 1))


def _loss_body(g_ref, m_ref, s_ref, loss_ref):
    ch = 512
    acc = jnp.zeros((1, ch), F32)
    for c in range(S // ch):
        sl = pl.ds(c * ch, ch)
        col = jax.lax.broadcasted_iota(jnp.int32, (ch, ch), 1) + c * ch
        trow = jax.lax.broadcasted_iota(jnp.int32, (ch, ch), 0) + c * ch
        lab = jnp.sum(jnp.where(col == trow, g_ref[sl, sl], 0.0),
                      axis=0, keepdims=True)
        lse = jnp.log(s_ref[:, sl]) + m_ref[:, sl]
        acc = acc + (lse - lab)
    loss_ref[...] = (jnp.sum(acc) / S).reshape(1, 1)


def _loss_kernel(gathered_rows, m, s):
    return pl.pallas_call(
        _loss_body,
        grid=(1,),
        in_specs=[
            pl.BlockSpec((S, S), lambda i: (0, 0)),
            pl.BlockSpec((1, S), lambda i: (0, 0)),
            pl.BlockSpec((1, S), lambda i: (0, 0)),
        ],
        out_specs=pl.BlockSpec((1, 1), lambda i: (0, 0)),
        out_shape=jax.ShapeDtypeStruct((1, 1), F32),
    )(gathered_rows, m, s)


def kernel(input_ids, labels, word_emb, pos_emb, type_emb, emb_ln_g, emb_ln_b,
           gate0, w1_0, b1_0, w2_0, b2_0, gate1, w1_1, b1_1, w2_1, b2_1,
           lm_w, lm_b):
    gathered = _sc_row_gather(word_emb, input_ids.reshape(S).astype(jnp.int32),
                              n_split=1)
    pos = jax.lax.slice(pos_emb, (2, 0), (2 + S, D))
    x = _embed_finish(gathered, pos, type_emb, emb_ln_g, emb_ln_b)
    x = _moe_block(x.astype(F32), gate0, w1_0, b1_0, w2_0, b2_0)
    x = _moe_block(x.astype(F32), gate1, w1_1, b1_1, w2_1, b2_1)
    scores_t, m, s = _lm_head(x.astype(F32), lm_w, lm_b)
    lab_rows = _sc_row_gather(scores_t, labels.reshape(S).astype(jnp.int32),
                              n_split=2)
    loss = _loss_kernel(lab_rows, m, s)
    return loss.reshape(()), scores_t.T.reshape(B, S, V)


# speedup vs baseline: 1.2268x; 1.2268x over previous
import functools

import jax
import jax.numpy as jnp
from jax.experimental import pallas as pl
from jax.experimental.pallas import tpu as pltpu
from jax.experimental.pallas import tpu_sc as plsc

V = 50265
D = 1024
E = 8
K = 5
DFF = 1024
B = 1
S = 2048

BF = jnp.bfloat16
F32 = jnp.float32

_NC = 2
_NS = 16
_NW = _NC * _NS


def _sc_row_gather(table, idx, n_split):
    n, d = len(idx), table.shape[1]
    bpw = n // _NW
    sub = bpw // n_split
    mesh = plsc.VectorSubcoreMesh(core_axis_name="c", subcore_axis_name="s")

    @functools.partial(
        pl.kernel,
        mesh=mesh,
        out_type=jax.ShapeDtypeStruct((n, d), table.dtype),
        scratch_types=[
            pltpu.VMEM((sub,), jnp.int32),
            pltpu.VMEM((sub, d), table.dtype),
            pltpu.SemaphoreType.DMA,
        ],
    )
    def k(tab_hbm, idx_hbm, out_hbm, idx_v, rows_v, sem):
        wid = jax.lax.axis_index("s") * _NC + jax.lax.axis_index("c")

        @pl.loop(0, n_split)
        def _(i):
            base = wid * bpw + i * sub
            pltpu.sync_copy(idx_hbm.at[pl.ds(base, sub)], idx_v)
            pltpu.async_copy(tab_hbm.at[idx_v], rows_v, sem).wait()
            pltpu.sync_copy(rows_v, out_hbm.at[pl.ds(base, sub)])

    return k(table, idx)


def _embed_body(g_ref, p_ref, t_ref, gam_ref, bet_ref, o_ref):
    h = g_ref[...] + p_ref[...] + t_ref[...]
    m = jnp.mean(h, axis=-1, keepdims=True)
    v = jnp.mean((h - m) ** 2, axis=-1, keepdims=True)
    x = (h - m) / jnp.sqrt(v + 1e-5) * gam_ref[...] + bet_ref[...]
    o_ref[...] = x.astype(BF).astype(F32)


def _embed_finish(gathered, pos, type_emb, gamma, beta):
    tb = 256
    return pl.pallas_call(
        _embed_body,
        grid=(S // tb,),
        in_specs=[
            pl.BlockSpec((tb, D), lambda i: (i, 0)),
            pl.BlockSpec((tb, D), lambda i: (i, 0)),
            pl.BlockSpec((1, D), lambda i: (0, 0)),
            pl.BlockSpec((1, D), lambda i: (0, 0)),
            pl.BlockSpec((1, D), lambda i: (0, 0)),
        ],
        out_specs=pl.BlockSpec((tb, D), lambda i: (i, 0)),
        out_shape=jax.ShapeDtypeStruct((S, D), F32),
    )(gathered, pos, type_emb, gamma.reshape(1, D), beta.reshape(1, D))


def _moe_body(x_ref, gate_ref, w1_ref, b1_ref, w2_ref, b2_ref, o_ref,
              wsel_ref, acc_ref):
    e = pl.program_id(0)

    @pl.when(e == 0)
    def _():
        l = jnp.dot(x_ref[...], gate_ref[...], preferred_element_type=F32)
        rank = jnp.zeros((S, E), dtype=jnp.int32)
        eidx = jax.lax.broadcasted_iota(jnp.int32, (S, E), 1)
        for j in range(E):
            lj = l[:, j:j + 1]
            beats = (lj > l) | ((lj == l) & (j < eidx))
            rank = rank + beats.astype(jnp.int32)
        sel = rank < K
        lm = jnp.where(sel, l, -jnp.inf)
        mx = jnp.max(lm, axis=1, keepdims=True)
        ex = jnp.where(sel, jnp.exp(l - mx), 0.0)
        wsel_ref[...] = (ex / jnp.sum(ex, axis=1, keepdims=True)).astype(BF)
        acc_ref[...] = jnp.zeros_like(acc_ref)

    ch = 512
    for c in range(S // ch):
        sl = pl.ds(c * ch, ch)
        h = jnp.dot(x_ref[sl, :], w1_ref[0], preferred_element_type=F32)
        h = jax.nn.gelu(h + b1_ref[0])
        y = jnp.dot(h, w2_ref[0], preferred_element_type=F32)
        y = (y + b2_ref[0]).astype(BF).astype(F32)
        eidx2 = jax.lax.broadcasted_iota(jnp.int32, (ch, E), 1)
        w_e = jnp.sum(
            jnp.where(eidx2 == e, wsel_ref[sl, :].astype(F32), 0.0),
            axis=1, keepdims=True)
        acc_ref[sl, :] += y * w_e

    @pl.when(e == E - 1)
    def _():
        o_ref[...] = acc_ref[...].astype(BF).astype(F32)


def _moe_block(x, gate, w1, b1, w2, b2):
    return pl.pallas_call(
        _moe_body,
        grid=(E,),
        in_specs=[
            pl.BlockSpec((S, D), lambda e: (0, 0)),
            pl.BlockSpec((D, E), lambda e: (0, 0)),
            pl.BlockSpec((1, D, DFF), lambda e: (e, 0, 0)),
            pl.BlockSpec((1, 1, DFF), lambda e: (e, 0, 0)),
            pl.BlockSpec((1, DFF, D), lambda e: (e, 0, 0)),
            pl.BlockSpec((1, 1, D), lambda e: (e, 0, 0)),
        ],
        out_specs=pl.BlockSpec((S, D), lambda e: (0, 0)),
        out_shape=jax.ShapeDtypeStruct((S, D), F32),
        scratch_shapes=[
            pltpu.VMEM((S, E), BF),
            pltpu.VMEM((S, D), F32),
        ],
    )(x, gate, w1, b1.reshape(E, 1, DFF), w2, b2.reshape(E, 1, D))


_VT = 1024
_NV = (V + _VT - 1) // _VT


def _lm_body(x_ref, w_ref, b_ref, sc_ref, m_out, s_out, m_ref, s_ref):
    v = pl.program_id(0)
    ch = 512

    def tile(c, masked):
        sl = pl.ds(c * ch, ch)
        sc = jax.lax.dot_general(
            w_ref[...], x_ref[sl, :],
            dimension_numbers=(((1,), (1,)), ((), ())),
            preferred_element_type=F32)
        sc = sc + b_ref[...]
        sc_ref[:, sl] = sc
        if masked:
            row = jax.lax.broadcasted_iota(jnp.int32, (_VT, ch), 0) + v * _VT
            scm = jnp.where(row < V, sc, -jnp.inf)
        else:
            scm = sc
        tmax = jnp.max(scm, axis=0, keepdims=True)

        @pl.when(v == 0)
        def _():
            m_ref[:, sl] = tmax
            s_ref[:, sl] = jnp.sum(jnp.exp(scm - tmax), axis=0, keepdims=True)

        @pl.when(v > 0)
        def _():
            m_old = m_ref[:, sl]
            m_new = jnp.maximum(m_old, tmax)
            s_ref[:, sl] = (s_ref[:, sl] * jnp.exp(m_old - m_new)
                            + jnp.sum(jnp.exp(scm - m_new), axis=0,
                                      keepdims=True))
            m_ref[:, sl] = m_new

    @pl.when(v < _NV - 1)
    def _():
        for c in range(S // ch):
            tile(c, masked=False)

    @pl.when(v == _NV - 1)
    def _():
        for c in range(S // ch):
            tile(c, masked=True)
        m_out[...] = m_ref[...]
        s_out[...] = s_ref[...]


def _lm_head(x, lm_w, lm_b):
    return pl.pallas_call(
        _lm_body,
        grid=(_NV,),
        in_specs=[
            pl.BlockSpec((S, D), lambda v: (0, 0)),
            pl.BlockSpec((_VT, D), lambda v: (v, 0)),
            pl.BlockSpec((_VT, 1), lambda v: (v, 0)),
        ],
        out_specs=[
            pl.BlockSpec((_VT, S), lambda v: (v, 0)),
            pl.BlockSpec((1, S), lambda v: (0, 0)),
            pl.BlockSpec((1, S), lambda v: (0, 0)),
        ],
        out_shape=[
            jax.ShapeDtypeStruct((V, S), F32),
            jax.ShapeDtypeStruct((1, S), F32),
            jax.ShapeDtypeStruct((1, S), F32),
        ],
        scratch_shapes=[
            pltpu.VMEM((1, S), F32),
            pltpu.VMEM((1, S), F32),
        ],
    )(x, lm_w.T, lm_b.reshape(V, 1))


def _loss_body(g_ref, m_ref, s_ref, loss_ref):
    ch = 512
    acc = jnp.zeros((1, ch), F32)
    for c in range(S // ch):
        sl = pl.ds(c * ch, ch)
        col = jax.lax.broadcasted_iota(jnp.int32, (ch, ch), 1) + c * ch
        trow = jax.lax.broadcasted_iota(jnp.int32, (ch, ch), 0) + c * ch
        lab = jnp.sum(jnp.where(col == trow, g_ref[sl, sl], 0.0),
                      axis=0, keepdims=True)
        lse = jnp.log(s_ref[:, sl]) + m_ref[:, sl]
        acc = acc + (lse - lab)
    loss_ref[...] = (jnp.sum(acc) / S).reshape(1, 1)


def _loss_kernel(gathered_rows, m, s):
    return pl.pallas_call(
        _loss_body,
        grid=(1,),
        in_specs=[
            pl.BlockSpec((S, S), lambda i: (0, 0)),
            pl.BlockSpec((1, S), lambda i: (0, 0)),
            pl.BlockSpec((1, S), lambda i: (0, 0)),
        ],
        out_specs=pl.BlockSpec((1, 1), lambda i: (0, 0)),
        out_shape=jax.ShapeDtypeStruct((1, 1), F32),
    )(gathered_rows, m, s)


def kernel(input_ids, labels, word_emb, pos_emb, type_emb, emb_ln_g, emb_ln_b,
           gate0, w1_0, b1_0, w2_0, b2_0, gate1, w1_1, b1_1, w2_1, b2_1,
           lm_w, lm_b):
    gathered = _sc_row_gather(word_emb, input_ids.reshape(S).astype(jnp.int32),
                              n_split=1)
    pos = jax.lax.slice(pos_emb, (2, 0), (2 + S, D))
    x = _embed_finish(gathered, pos, type_emb, emb_ln_g, emb_ln_b)
    x = _moe_block(x, gate0, w1_0, b1_0, w2_0, b2_0)
    x = _moe_block(x, gate1, w1_1, b1_1, w2_1, b2_1)
    scores_t, m, s = _lm_head(x, lm_w, lm_b)
    lab_rows = _sc_row_gather(scores_t, labels.reshape(S).astype(jnp.int32),
                              n_split=2)
    loss = _loss_kernel(lab_rows, m, s)
    return loss.reshape(()), scores_t.T.reshape(B, S, V)
